# baseline (speedup 1.0000x reference)
import math

import jax
import jax.numpy as jnp
from jax.experimental import pallas as pl
from jax.experimental.pallas import tpu as pltpu

_GROUP = 4
_BUDGET = 0.5
_NBUF = 4
_AHEAD = 3


def _body(sal_hbm, msk_ref, wt_ref, b_ref, out_ref, lasso_ref,
          buf_ref, acc_ref, mask_acc_ref, sems,
          *, n_c, c_blk, n_b, n_ch, n_px, k_drop):
    bi = pl.program_id(0)
    ci = pl.program_id(1)
    t = bi * n_c + ci
    n_t = n_b * n_c

    def chunk_copy(u):
        ub = u // n_c
        uc = u % n_c
        return pltpu.make_async_copy(
            sal_hbm.at[ub, pl.ds(uc * c_blk, c_blk)],
            buf_ref.at[jax.lax.rem(u, _NBUF)],
            sems.at[jax.lax.rem(u, _NBUF)])

    @pl.when(t == 0)
    def _prologue():
        for u in range(_AHEAD):
            chunk_copy(jnp.int32(u)).start()

    @pl.when(t + _AHEAD < n_t)
    def _prefetch():
        chunk_copy(t + _AHEAD).start()

    chunk_copy(t).wait()

    sal = buf_ref[jax.lax.rem(t, _NBUF)]
    m = msk_ref[0]
    part = jnp.sum(sal * m, axis=1)
    acc_ref[bi, pl.ds(ci * c_blk, c_blk)] = part

    @pl.when(ci == 0)
    def _mask_sum():
        mask_acc_ref[pl.ds(bi, 1), :] = jnp.sum(m[0], axis=0, keepdims=True)

    @pl.when(t == n_t - 1)
    def _finalize():
        total = float(n_px)
        pooled = jnp.sum(acc_ref[:], axis=2) / total
        active = jnp.sum(mask_acc_ref[:], axis=1, keepdims=True) + 0.0001
        pooled = pooled * total / active
        z = jax.nn.sigmoid(
            jnp.dot(pooled, wt_ref[:], preferred_element_type=jnp.float32)
            + b_ref[:])
        lasso_ref[:] = jnp.full((1, 1), jnp.mean(jnp.sum(z, axis=-1)),
                                jnp.float32)

        zi = z[:, :, None]
        zj = z[:, None, :]
        ii = jax.lax.broadcasted_iota(jnp.int32, (n_b, n_ch, n_ch), 1)
        jj = jax.lax.broadcasted_iota(jnp.int32, (n_b, n_ch, n_ch), 2)
        below = jnp.logical_or(zj < zi,
                               jnp.logical_and(zj == zi, jj < ii))
        cnt = jnp.sum(below.astype(jnp.int32), axis=2)
        keep = jnp.logical_and(cnt >= k_drop, z > 0)

        n_out = n_ch * _GROUP
        row = jax.lax.broadcasted_iota(jnp.int32, (n_ch, n_out), 0)
        col = jax.lax.broadcasted_iota(jnp.int32, (n_ch, n_out), 1)
        expand = (row == col // _GROUP).astype(jnp.float32)
        out_ref[:] = jnp.dot(keep.astype(jnp.float32), expand,
                             preferred_element_type=jnp.float32
                             ).astype(jnp.int32)


def kernel(x, saliency_mask, mask_hard, W, b):
    B, C, H, Wd = saliency_mask.shape
    F = W.shape[0]
    k_drop = math.ceil((1.0 - _BUDGET) * F)

    c_blk = 32
    n_c = C // c_blk

    wt = W.T
    b2 = b.reshape(1, F)

    expanded, lasso = pl.pallas_call(
        lambda *refs: _body(*refs, n_c=n_c, c_blk=c_blk, n_b=B, n_ch=F,
                            n_px=H * Wd, k_drop=k_drop),
        grid=(B, n_c),
        in_specs=[
            pl.BlockSpec(memory_space=pltpu.HBM),
            pl.BlockSpec((1, 1, H, Wd), lambda bi, ci: (bi, 0, 0, 0)),
            pl.BlockSpec((C, F), lambda bi, ci: (0, 0)),
            pl.BlockSpec((1, F), lambda bi, ci: (0, 0)),
        ],
        out_specs=[
            pl.BlockSpec((B, F * _GROUP), lambda bi, ci: (0, 0)),
            pl.BlockSpec((1, 1), lambda bi, ci: (0, 0)),
        ],
        out_shape=[
            jax.ShapeDtypeStruct((B, F * _GROUP), jnp.int32),
            jax.ShapeDtypeStruct((1, 1), jnp.float32),
        ],
        scratch_shapes=[
            pltpu.VMEM((_NBUF, c_blk, H, Wd), jnp.float32),
            pltpu.VMEM((B, C, Wd), jnp.float32),
            pltpu.VMEM((B, Wd), jnp.float32),
            pltpu.SemaphoreType.DMA((_NBUF,)),
        ],
    )(saliency_mask, mask_hard, wt, b2)

    return expanded, lasso.reshape(())


# speedup vs baseline: 1.0073x; 1.0073x over previous
import math

import jax
import jax.numpy as jnp
from jax.experimental import pallas as pl
from jax.experimental.pallas import tpu as pltpu

_GROUP = 4
_BUDGET = 0.5


def _pool_body(sal_ref, msk_ref, part_ref, mpart_ref):
    ci = pl.program_id(1)
    sal = sal_ref[0]
    m = msk_ref[0]
    part_ref[0] = jnp.sum(sal * m, axis=1)

    @pl.when(ci == 0)
    def _mask_sum():
        mpart_ref[0] = jnp.sum(m[0], axis=0, keepdims=True)


def _tail_body(part_ref, mpart_ref, wt_ref, b_ref, out_ref, lasso_ref,
               *, n_b, n_ch, n_px, k_drop):
    total = float(n_px)
    pooled = jnp.sum(part_ref[:], axis=2) / total
    active = jnp.sum(mpart_ref[:], axis=2)[:, 0:1] + 0.0001
    pooled = pooled * total / active
    z = jax.nn.sigmoid(
        jnp.dot(pooled, wt_ref[:], preferred_element_type=jnp.float32)
        + b_ref[:])
    lasso_ref[:] = jnp.full((1, 1), jnp.mean(jnp.sum(z, axis=-1)),
                            jnp.float32)

    zi = z[:, :, None]
    zj = z[:, None, :]
    ii = jax.lax.broadcasted_iota(jnp.int32, (n_b, n_ch, n_ch), 1)
    jj = jax.lax.broadcasted_iota(jnp.int32, (n_b, n_ch, n_ch), 2)
    below = jnp.logical_or(zj < zi,
                           jnp.logical_and(zj == zi, jj < ii))
    cnt = jnp.sum(below.astype(jnp.int32), axis=2)
    keep = jnp.logical_and(cnt >= k_drop, z > 0)

    n_out = n_ch * _GROUP
    row = jax.lax.broadcasted_iota(jnp.int32, (n_ch, n_out), 0)
    col = jax.lax.broadcasted_iota(jnp.int32, (n_ch, n_out), 1)
    expand = (row == col // _GROUP).astype(jnp.float32)
    out_ref[:] = jnp.dot(keep.astype(jnp.float32), expand,
                         preferred_element_type=jnp.float32
                         ).astype(jnp.int32)


def kernel(x, saliency_mask, mask_hard, W, b):
    B, C, H, Wd = saliency_mask.shape
    F = W.shape[0]
    k_drop = math.ceil((1.0 - _BUDGET) * F)

    c_blk = 64
    n_c = C // c_blk

    parts, mparts = pl.pallas_call(
        _pool_body,
        grid=(B, n_c),
        in_specs=[
            pl.BlockSpec((1, c_blk, H, Wd), lambda bi, ci: (bi, ci, 0, 0)),
            pl.BlockSpec((1, 1, H, Wd), lambda bi, ci: (bi, 0, 0, 0)),
        ],
        out_specs=[
            pl.BlockSpec((1, c_blk, Wd), lambda bi, ci: (bi, ci, 0)),
            pl.BlockSpec((1, 1, Wd), lambda bi, ci: (bi, 0, 0)),
        ],
        out_shape=[
            jax.ShapeDtypeStruct((B, C, Wd), jnp.float32),
            jax.ShapeDtypeStruct((B, 1, Wd), jnp.float32),
        ],
    )(saliency_mask, mask_hard)

    expanded, lasso = pl.pallas_call(
        lambda *refs: _tail_body(*refs, n_b=B, n_ch=F, n_px=H * Wd,
                                 k_drop=k_drop),
        out_shape=[
            jax.ShapeDtypeStruct((B, F * _GROUP), jnp.int32),
            jax.ShapeDtypeStruct((1, 1), jnp.float32),
        ],
    )(parts, mparts, W.T, b.reshape(1, F))

    return expanded, lasso.reshape(())


# speedup vs baseline: 1.0301x; 1.0226x over previous
import functools
import math

import jax
import jax.numpy as jnp
from jax import lax
from jax.experimental import pallas as pl
from jax.experimental.pallas import tpu as pltpu
from jax.experimental.pallas import tpu_sc as plsc

_GROUP = 4
_BUDGET = 0.5

_NC = 2
_NS = 16
_L = 16
_G = 8
_PC = 3584


def _sc_pool(sal_ref, msk_ref, parts_ref, msums_ref,
             buf_ref, mask_buf, res_buf, msum_buf, sem_a, sem_b,
             *, n_b, n_ch, n_px):
    n_pc = n_px // _PC
    w_per_b = (_NC * _NS) // n_b
    cpw = n_ch // w_per_b
    n_g = cpw // _G
    n_t = n_g * n_pc

    wid = lax.axis_index("s") * _NC + lax.axis_index("c")
    bi = wid // w_per_b
    cbase = (wid % w_per_b) * cpw

    pltpu.sync_copy(msk_ref.at[bi], mask_buf)

    def sal_copy(t, slot):
        g = t // n_pc
        pc = t - g * n_pc
        return pltpu.make_async_copy(
            sal_ref.at[bi, pl.ds(cbase + g * _G, _G), pl.ds(pc * _PC, _PC)],
            buf_ref.at[slot],
            sem_a if slot == 0 else sem_b)

    def zero_res(c, carry):
        res_buf[c] = jnp.zeros((_L,), jnp.float32)
        return carry
    lax.fori_loop(0, cpw, zero_res, 0)

    def tile_compute(t, slot):
        g = t // n_pc
        pc = t - g * n_pc
        moff = pc * _PC

        def body(i, accs):
            m = mask_buf[pl.ds(moff + i * _L, _L)]
            return tuple(
                accs[k] + buf_ref[slot, k, pl.ds(i * _L, _L)] * m
                for k in range(_G))

        accs = lax.fori_loop(
            0, _PC // _L, body,
            tuple(jnp.zeros((_L,), jnp.float32) for _ in range(_G)))
        c0 = g * _G
        for k in range(_G):
            res_buf[c0 + k] = res_buf[c0 + k] + accs[k]

    sal_copy(jnp.int32(0), 0).start()

    def pair(i, carry):
        t0 = 2 * i
        sal_copy(t0 + 1, 1).start()
        sal_copy(t0, 0).wait()
        tile_compute(t0, 0)

        @pl.when(t0 + 2 < n_t)
        def _():
            sal_copy(t0 + 2, 0).start()
        sal_copy(t0 + 1, 1).wait()
        tile_compute(t0 + 1, 1)
        return carry

    lax.fori_loop(0, n_t // 2, pair, 0)

    pltpu.sync_copy(res_buf, parts_ref.at[bi, pl.ds(cbase, cpw)])

    @pl.when(wid % w_per_b == 0)
    def _mask_total():
        def msum_body(i, acc):
            return acc + mask_buf[pl.ds(i * _L, _L)]
        msum = lax.fori_loop(0, n_px // _L, msum_body,
                             jnp.zeros((_L,), jnp.float32))
        msum_buf[...] = msum
        pltpu.sync_copy(msum_buf, msums_ref.at[bi])


def _tail_body(part_ref, msum_ref, wt_ref, b_ref, out_ref, lasso_ref,
               *, n_b, n_ch, n_px, k_drop):
    total = float(n_px)
    pooled = jnp.sum(part_ref[:], axis=2) / total
    active = jnp.sum(msum_ref[:], axis=1, keepdims=True) + 0.0001
    pooled = pooled * total / active
    z = jax.nn.sigmoid(
        jnp.dot(pooled, wt_ref[:], preferred_element_type=jnp.float32)
        + b_ref[:])
    lasso_ref[:] = jnp.full((1, 1), jnp.mean(jnp.sum(z, axis=-1)),
                            jnp.float32)

    zi = z[:, :, None]
    zj = z[:, None, :]
    ii = lax.broadcasted_iota(jnp.int32, (n_b, n_ch, n_ch), 1)
    jj = lax.broadcasted_iota(jnp.int32, (n_b, n_ch, n_ch), 2)
    below = jnp.logical_or(zj < zi,
                           jnp.logical_and(zj == zi, jj < ii))
    cnt = jnp.sum(below.astype(jnp.int32), axis=2)
    keep = jnp.logical_and(cnt >= k_drop, z > 0)

    n_out = n_ch * _GROUP
    row = lax.broadcasted_iota(jnp.int32, (n_ch, n_out), 0)
    col = lax.broadcasted_iota(jnp.int32, (n_ch, n_out), 1)
    expand = (row == col // _GROUP).astype(jnp.float32)
    out_ref[:] = jnp.dot(keep.astype(jnp.float32), expand,
                         preferred_element_type=jnp.float32
                         ).astype(jnp.int32)


def kernel(x, saliency_mask, mask_hard, W, b):
    B, C, H, Wd = saliency_mask.shape
    S = H * Wd
    F = W.shape[0]
    k_drop = math.ceil((1.0 - _BUDGET) * F)

    sal = saliency_mask.reshape(B, C, S)
    msk = mask_hard.reshape(B, S)

    mesh = plsc.VectorSubcoreMesh(core_axis_name="c", subcore_axis_name="s",
                                  num_cores=_NC, num_subcores=_NS)
    sc_pool = functools.partial(
        pl.kernel,
        out_type=[
            jax.ShapeDtypeStruct((B, C, _L), jnp.float32),
            jax.ShapeDtypeStruct((B, _L), jnp.float32),
        ],
        mesh=mesh,
        scratch_types=[
            pltpu.VMEM((2, _G, _PC), jnp.float32),
            pltpu.VMEM((S,), jnp.float32),
            pltpu.VMEM((C // ((_NC * _NS) // B), _L), jnp.float32),
            pltpu.VMEM((_L,), jnp.float32),
            pltpu.SemaphoreType.DMA,
            pltpu.SemaphoreType.DMA,
        ],
    )(functools.partial(_sc_pool, n_b=B, n_ch=C, n_px=S))

    parts, msums = sc_pool(sal, msk)

    expanded, lasso = pl.pallas_call(
        lambda *refs: _tail_body(*refs, n_b=B, n_ch=F, n_px=S,
                                 k_drop=k_drop),
        out_shape=[
            jax.ShapeDtypeStruct((B, F * _GROUP), jnp.int32),
            jax.ShapeDtypeStruct((1, 1), jnp.float32),
        ],
    )(parts, msums, W.T, b.reshape(1, F))

    return expanded, lasso.reshape(())
